# baseline (speedup 1.0000x reference)
import functools
import jax
import jax.numpy as jnp
from jax import lax
from jax.experimental import pallas as pl
from jax.experimental.pallas import tpu as pltpu
from jax.experimental.pallas import tpu_sc as plsc

N = 320000
CH = 128
NCB = CH // 16
NSEG = 2048
NW = 32
SPW = NSEG // NW
TB = 256
NGRP = TB // 16
STEPS = 3

_mesh = plsc.VectorSubcoreMesh(core_axis_name="c", subcore_axis_name="s")


@functools.partial(
    pl.kernel,
    out_type=jax.ShapeDtypeStruct((NW, SPW, CH), jnp.float32),
    mesh=_mesh,
    scratch_types=[
        pltpu.VMEM((TB, CH), jnp.float32),
        pltpu.VMEM((TB, CH), jnp.float32),
        pltpu.VMEM((TB,), jnp.int32),
        pltpu.VMEM((TB,), jnp.int32),
        pltpu.VMEM((SPW, CH), jnp.float32),
        pltpu.VMEM((SPW, CH), jnp.float32),
        pltpu.VMEM((SPW, CH), jnp.float32),
        pltpu.VMEM((SPW, 16), jnp.float32),
        pltpu.VMEM((SPW, 16), jnp.float32),
        pltpu.VMEM((48,), jnp.int32),
        pltpu.SemaphoreType.DMA,
        pltpu.SemaphoreType.DMA,
        pltpu.SemaphoreType.DMA,
        pltpu.SemaphoreType.DMA,
    ],
    compiler_params=pltpu.CompilerParams(needs_layout_passes=False),
)
def _attend(x_hbm, b_hbm, q_hbm, off_hbm, out_hbm,
            x_v0, x_v1, b_v0, b_v1, q_v, R_a, R_b, Z_a, Z_b, off_v,
            sx0, sx1, sb0, sb1):
    wid = lax.axis_index("s") * 2 + lax.axis_index("c")
    sb = pl.multiple_of(wid * SPW, SPW)
    iota = lax.iota(jnp.int32, 16)
    zero16 = jnp.zeros((16,), jnp.float32)

    pltpu.sync_copy(off_hbm, off_v)
    offs = plsc.load_gather(
        off_v, [jnp.full((16,), wid, jnp.int32) + jnp.minimum(iota, 1)])
    r0 = offs[0]
    r1 = offs[1]
    pltpu.sync_copy(q_hbm.at[pl.ds(sb, SPW), :], q_v)

    def zbody(s, carry):
        Z_a[s, pl.ds(0, 16)] = zero16
        Z_b[s, pl.ds(0, 16)] = zero16
        for cb in range(NCB):
            R_a[s, pl.ds(cb * 16, 16)] = zero16
            R_b[s, pl.ds(cb * 16, 16)] = zero16
        return carry

    lax.fori_loop(0, SPW, zbody, 0)

    t0 = r0 // TB
    t1 = (r1 + TB - 1) // TB

    bufs = ((x_v0, b_v0, sx0, sb0), (x_v1, b_v1, sx1, sb1))

    def issue(t, bi):
        x_b, b_b, sx, sbm = bufs[bi]
        ts = pl.multiple_of(t * TB, TB)
        pltpu.async_copy(x_hbm.at[pl.ds(ts, TB), :], x_b, sx)
        pltpu.async_copy(b_hbm.at[pl.ds(ts, TB)], b_b, sbm)

    def wait(bi):
        x_b, b_b, sx, sbm = bufs[bi]
        pltpu.make_async_copy(x_hbm.at[pl.ds(0, TB), :], x_b, sx).wait()
        pltpu.make_async_copy(b_hbm.at[pl.ds(0, TB)], b_b, sbm).wait()

    def process(t, bi):
        x_b, b_b, _, _ = bufs[bi]
        ts = pl.multiple_of(t * TB, TB)

        @plsc.parallel_loop(0, NGRP, unroll=1)
        def grp(g):
            row0 = g * 16
            b16 = b_b[pl.ds(row0, 16)]
            rowid = ts + row0 + iota
            validf = jnp.where((rowid >= r0) & (rowid < r1), 1.0, 0.0)
            brel = jnp.clip(b16 - sb, 0, SPW - 1)

            def rows(qrow_of):
                for k in range(16):
                    bk = brel[k]
                    qr = qrow_of(bk)
                    xr = [x_b[row0 + k, pl.ds(cb * 16, 16)]
                          for cb in range(NCB)]
                    acc0 = xr[0] * qr[0]
                    acc1 = xr[1] * qr[1]
                    for cb in range(2, NCB, 2):
                        acc0 = acc0 + xr[cb] * qr[cb]
                        acc1 = acc1 + xr[cb + 1] * qr[cb + 1]
                    e_s = jnp.sum(acc0 + acc1)
                    wv = validf[k] * jnp.exp(jnp.full((16,), e_s, jnp.float32))
                    Rt = R_a if k % 2 == 0 else R_b
                    Zt = Z_a if k % 2 == 0 else Z_b
                    plsc.addupdate(Zt.at[bk, pl.ds(0, 16)], wv)
                    for cb in range(NCB):
                        plsc.addupdate(Rt.at[bk, pl.ds(cb * 16, 16)],
                                       wv * xr[cb])

            b0 = brel[0]
            hom = jnp.all(brel == jnp.full((16,), b0, jnp.int32))

            @pl.when(hom)
            def _():
                qc = [q_v[b0, pl.ds(cb * 16, 16)] for cb in range(NCB)]
                rows(lambda bk: qc)

            @pl.when(jnp.logical_not(hom))
            def _():
                rows(lambda bk: [q_v[bk, pl.ds(cb * 16, 16)]
                                 for cb in range(NCB)])

    @pl.when(t0 < t1)
    def _():
        issue(t0, 0)

    @pl.when(t0 + 1 < t1)
    def _():
        issue(t0 + 1, 1)

    npairs = (t1 - t0 + 1) // 2

    def pair(i, carry):
        tp = t0 + 2 * i
        wait(0)
        process(tp, 0)

        @pl.when(tp + 2 < t1)
        def _():
            issue(tp + 2, 0)

        @pl.when(tp + 1 < t1)
        def _():
            wait(1)
            process(tp + 1, 1)

            @pl.when(tp + 3 < t1)
            def _():
                issue(tp + 3, 1)

        return carry

    lax.fori_loop(0, npairs, pair, 0)

    zero16i = jnp.zeros((16,), jnp.int32)
    for sv in range(SPW // 16):
        zv = (plsc.load_gather(Z_a, [sv * 16 + iota, zero16i])
              + plsc.load_gather(Z_b, [sv * 16 + iota, zero16i]))
        ziv = 1.0 / jnp.maximum(zv, 1e-10)
        for k in range(16):
            s = sv * 16 + k
            zk = ziv[k]
            for cb in range(NCB):
                sl = pl.ds(cb * 16, 16)
                R_a[s, sl] = (R_a[s, sl] + R_b[s, sl]) * zk

    pltpu.sync_copy(R_a, out_hbm.at[wid])


def _lstm_body(q_ref, c_ref, r_ref, Wqh_ref, Wr_ref, bias_ref,
               qo_ref, co_ref):
    q = q_ref[...]
    c = c_ref[...]
    gates = jnp.dot(q, Wqh_ref[...], preferred_element_type=jnp.float32)
    gates = gates + jnp.dot(r_ref[...], Wr_ref[...],
                            preferred_element_type=jnp.float32)
    gates = gates + bias_ref[...]
    i = jax.nn.sigmoid(gates[:, :CH])
    f = jax.nn.sigmoid(gates[:, CH:2 * CH])
    g = jnp.tanh(gates[:, 2 * CH:3 * CH])
    o = jax.nn.sigmoid(gates[:, 3 * CH:])
    c_new = f * c + i * g
    co_ref[...] = c_new
    qo_ref[...] = o * jnp.tanh(c_new)


_lstm = pl.pallas_call(
    _lstm_body,
    out_shape=[
        jax.ShapeDtypeStruct((NSEG, CH), jnp.float32),
        jax.ShapeDtypeStruct((NSEG, CH), jnp.float32),
    ],
)


def kernel(x, batch, W_ih, W_hh, b_ih, b_hh):
    b32 = batch.astype(jnp.int32)
    off = jnp.searchsorted(
        b32, jnp.arange(0, NSEG + 1, SPW, dtype=jnp.int32)).astype(jnp.int32)
    off = jnp.concatenate([off, jnp.zeros((48 - NSEG // SPW - 1,), jnp.int32)])
    Wqh = W_ih[:, :CH].T + W_hh.T
    Wr = W_ih[:, CH:].T
    bias = (b_ih + b_hh)[None, :]

    q = jnp.zeros((NSEG, CH), jnp.float32)
    c = jnp.zeros((NSEG, CH), jnp.float32)
    r = jnp.zeros((NSEG, CH), jnp.float32)
    for _ in range(STEPS):
        q, c = _lstm(q, c, r, Wqh, Wr, bias)
        r = _attend(x, b32, q, off).reshape(NSEG, CH)
    return jnp.concatenate([q, r], axis=-1)


# speedup vs baseline: 1.6126x; 1.6126x over previous
import functools
import jax
import jax.numpy as jnp
from jax import lax
from jax.experimental import pallas as pl
from jax.experimental.pallas import tpu as pltpu
from jax.experimental.pallas import tpu_sc as plsc

N = 320000
CH = 128
NCB = CH // 16
NSEG = 2048
NW = 32
SPW = NSEG // NW
TB = 256
NGRP = TB // 16
STEPS = 3

_mesh = plsc.VectorSubcoreMesh(core_axis_name="c", subcore_axis_name="s")


@functools.partial(
    pl.kernel,
    out_type=jax.ShapeDtypeStruct((NW, SPW, CH), jnp.float32),
    mesh=_mesh,
    scratch_types=[
        pltpu.VMEM((TB, CH), jnp.float32),
        pltpu.VMEM((TB, CH), jnp.float32),
        pltpu.VMEM((TB,), jnp.int32),
        pltpu.VMEM((TB,), jnp.int32),
        pltpu.VMEM((SPW, CH), jnp.float32),
        pltpu.VMEM((SPW, CH), jnp.float32),
        pltpu.VMEM((SPW, CH), jnp.float32),
        pltpu.VMEM((SPW, 16), jnp.float32),
        pltpu.VMEM((SPW, 16), jnp.float32),
        pltpu.VMEM((48,), jnp.int32),
        pltpu.SemaphoreType.DMA,
        pltpu.SemaphoreType.DMA,
        pltpu.SemaphoreType.DMA,
        pltpu.SemaphoreType.DMA,
    ],
    compiler_params=pltpu.CompilerParams(needs_layout_passes=False),
)
def _attend(x_hbm, b_hbm, q_hbm, off_hbm, out_hbm,
            x_v0, x_v1, b_v0, b_v1, q_v, R_a, R_b, Z_a, Z_b, off_v,
            sx0, sx1, sb0, sb1):
    wid = lax.axis_index("s") * 2 + lax.axis_index("c")
    sb = pl.multiple_of(wid * SPW, SPW)
    iota = lax.iota(jnp.int32, 16)
    zero16 = jnp.zeros((16,), jnp.float32)

    pltpu.sync_copy(off_hbm, off_v)
    offs = plsc.load_gather(
        off_v, [jnp.full((16,), wid, jnp.int32) + jnp.minimum(iota, 1)])
    r0 = offs[0]
    r1 = offs[1]
    pltpu.sync_copy(q_hbm.at[pl.ds(sb, SPW), :], q_v)

    def zbody(s, carry):
        Z_a[s, pl.ds(0, 16)] = zero16
        Z_b[s, pl.ds(0, 16)] = zero16
        for cb in range(NCB):
            R_a[s, pl.ds(cb * 16, 16)] = zero16
            R_b[s, pl.ds(cb * 16, 16)] = zero16
        return carry

    lax.fori_loop(0, SPW, zbody, 0)

    t0 = r0 // TB
    t1 = (r1 + TB - 1) // TB

    bufs = ((x_v0, b_v0, sx0, sb0), (x_v1, b_v1, sx1, sb1))

    def issue(t, bi):
        x_b, b_b, sx, sbm = bufs[bi]
        ts = pl.multiple_of(t * TB, TB)
        pltpu.async_copy(x_hbm.at[pl.ds(ts, TB), :], x_b, sx)
        pltpu.async_copy(b_hbm.at[pl.ds(ts, TB)], b_b, sbm)

    def wait(bi):
        x_b, b_b, sx, sbm = bufs[bi]
        pltpu.make_async_copy(x_hbm.at[pl.ds(0, TB), :], x_b, sx).wait()
        pltpu.make_async_copy(b_hbm.at[pl.ds(0, TB)], b_b, sbm).wait()

    def process(t, bi):
        x_b, b_b, _, _ = bufs[bi]
        ts = pl.multiple_of(t * TB, TB)

        @plsc.parallel_loop(0, NGRP, unroll=1)
        def grp(g):
            row0 = g * 16
            b16 = b_b[pl.ds(row0, 16)]
            rowid = ts + row0 + iota
            validf = jnp.where((rowid >= r0) & (rowid < r1), 1.0, 0.0)
            brel = jnp.clip(b16 - sb, 0, SPW - 1)

            for k in range(16):
                bk = brel[k]
                xr = [x_b[row0 + k, pl.ds(cb * 16, 16)] for cb in range(NCB)]
                acc0 = xr[0] * q_v[bk, pl.ds(0, 16)]
                acc1 = xr[1] * q_v[bk, pl.ds(16, 16)]
                for cb in range(2, NCB, 2):
                    acc0 = acc0 + xr[cb] * q_v[bk, pl.ds(cb * 16, 16)]
                    acc1 = acc1 + xr[cb + 1] * q_v[bk, pl.ds(cb * 16 + 16, 16)]
                e_s = jnp.sum(acc0 + acc1)
                wv = validf[k] * jnp.exp(jnp.full((16,), e_s, jnp.float32))
                Rt = R_a if k % 2 == 0 else R_b
                Zt = Z_a if k % 2 == 0 else Z_b
                plsc.addupdate(Zt.at[bk, pl.ds(0, 16)], wv)
                for cb in range(NCB):
                    plsc.addupdate(Rt.at[bk, pl.ds(cb * 16, 16)],
                                   wv * xr[cb])

    @pl.when(t0 < t1)
    def _():
        issue(t0, 0)

    @pl.when(t0 + 1 < t1)
    def _():
        issue(t0 + 1, 1)

    npairs = (t1 - t0 + 1) // 2

    def pair(i, carry):
        tp = t0 + 2 * i
        wait(0)
        process(tp, 0)

        @pl.when(tp + 2 < t1)
        def _():
            issue(tp + 2, 0)

        @pl.when(tp + 1 < t1)
        def _():
            wait(1)
            process(tp + 1, 1)

            @pl.when(tp + 3 < t1)
            def _():
                issue(tp + 3, 1)

        return carry

    lax.fori_loop(0, npairs, pair, 0)

    zero16i = jnp.zeros((16,), jnp.int32)
    for sv in range(SPW // 16):
        zv = (plsc.load_gather(Z_a, [sv * 16 + iota, zero16i])
              + plsc.load_gather(Z_b, [sv * 16 + iota, zero16i]))
        ziv = 1.0 / jnp.maximum(zv, 1e-10)
        for k in range(16):
            s = sv * 16 + k
            zk = ziv[k]
            for cb in range(NCB):
                sl = pl.ds(cb * 16, 16)
                R_a[s, sl] = (R_a[s, sl] + R_b[s, sl]) * zk

    pltpu.sync_copy(R_a, out_hbm.at[wid])


def _lstm_body(q_ref, c_ref, r_ref, Wqh_ref, Wr_ref, bias_ref,
               qo_ref, co_ref):
    q = q_ref[...]
    c = c_ref[...]
    gates = jnp.dot(q, Wqh_ref[...], preferred_element_type=jnp.float32)
    gates = gates + jnp.dot(r_ref[...], Wr_ref[...],
                            preferred_element_type=jnp.float32)
    gates = gates + bias_ref[...]
    i = jax.nn.sigmoid(gates[:, :CH])
    f = jax.nn.sigmoid(gates[:, CH:2 * CH])
    g = jnp.tanh(gates[:, 2 * CH:3 * CH])
    o = jax.nn.sigmoid(gates[:, 3 * CH:])
    c_new = f * c + i * g
    co_ref[...] = c_new
    qo_ref[...] = o * jnp.tanh(c_new)


_lstm = pl.pallas_call(
    _lstm_body,
    out_shape=[
        jax.ShapeDtypeStruct((NSEG, CH), jnp.float32),
        jax.ShapeDtypeStruct((NSEG, CH), jnp.float32),
    ],
)


def kernel(x, batch, W_ih, W_hh, b_ih, b_hh):
    b32 = batch.astype(jnp.int32)
    off = jnp.searchsorted(
        b32, jnp.arange(0, NSEG + 1, SPW, dtype=jnp.int32)).astype(jnp.int32)
    off = jnp.concatenate([off, jnp.zeros((48 - NSEG // SPW - 1,), jnp.int32)])
    Wqh = W_ih[:, :CH].T + W_hh.T
    Wr = W_ih[:, CH:].T
    bias = (b_ih + b_hh)[None, :]

    q = jnp.zeros((NSEG, CH), jnp.float32)
    c = jnp.zeros((NSEG, CH), jnp.float32)
    r = jnp.zeros((NSEG, CH), jnp.float32)
    for _ in range(STEPS):
        q, c = _lstm(q, c, r, Wqh, Wr, bias)
        r = _attend(x, b32, q, off).reshape(NSEG, CH)
    return jnp.concatenate([q, r], axis=-1)
